# Initial kernel scaffold; baseline (speedup 1.0000x reference)
import jax
import jax.numpy as jnp
from jax.experimental import pallas as pl


def kernel(src_word, src_pos, W_word, W_pos):
    raise NotImplementedError("write your pallas kernel here")


# speedup vs baseline: 1.9843x; 1.9843x over previous
import functools

import jax
import jax.numpy as jnp
from jax import lax
from jax.experimental import pallas as pl
from jax.experimental.pallas import tpu as pltpu
from jax.experimental.pallas import tpu_sc as plsc

DIM = 64
SCALE = 8.0
LANES = 16
CHUNK = 128


@functools.lru_cache(maxsize=None)
def _build(n_rows: int):
    info = plsc.get_sparse_core_info()
    nw = info.num_cores * info.num_subcores
    rows_per_w = n_rows // nw
    n_chunks = rows_per_w // CHUNK
    assert rows_per_w % CHUNK == 0

    mesh = plsc.VectorSubcoreMesh(core_axis_name="c", subcore_axis_name="s")

    @functools.partial(
        pl.kernel,
        mesh=mesh,
        compiler_params=pltpu.CompilerParams(use_tc_tiling_on_sc=False),
        out_type=jax.ShapeDtypeStruct((n_rows, DIM), jnp.float32),
        scratch_types=[
            pltpu.VMEM((rows_per_w,), jnp.int32),
            pltpu.VMEM((rows_per_w,), jnp.int32),
            pltpu.VMEM((CHUNK, DIM), jnp.float32),
            pltpu.VMEM((CHUNK, DIM), jnp.float32),
            pltpu.SemaphoreType.DMA,
            pltpu.SemaphoreType.DMA,
        ],
    )
    def emb(w_word, w_pos, iw_hbm, ip_hbm, out_hbm,
            iw_v, ip_v, bw, bp, sem_w, sem_p):
        wid = lax.axis_index("s") * info.num_cores + lax.axis_index("c")
        base = wid * rows_per_w
        pltpu.sync_copy(iw_hbm.at[pl.ds(base, rows_per_w)], iw_v)
        pltpu.sync_copy(ip_hbm.at[pl.ds(base, rows_per_w)], ip_v)

        def chunk_body(c, carry):
            off = c * CHUNK
            cw = pltpu.async_copy(w_word.at[iw_v.at[pl.ds(off, CHUNK)]], bw, sem_w)
            cp = pltpu.async_copy(w_pos.at[ip_v.at[pl.ds(off, CHUNK)]], bp, sem_p)
            cw.wait()
            cp.wait()

            def row_body(r, rcarry):
                for k in range(4):
                    ri = r * 4 + k
                    for j in range(DIM // LANES):
                        s = pl.ds(j * LANES, LANES)
                        bp[ri, s] = bw[ri, s] * SCALE + bp[ri, s]
                return rcarry

            lax.fori_loop(0, CHUNK // 4, row_body, 0, unroll=False)
            pltpu.sync_copy(bp, out_hbm.at[pl.ds(base + off, CHUNK)])
            return carry

        lax.fori_loop(0, n_chunks, chunk_body, 0, unroll=False)

    return emb


def kernel(src_word, src_pos, W_word, W_pos):
    B, L = src_word.shape
    n = B * L
    iw = src_word.reshape(-1).astype(jnp.int32)
    ip = src_pos.reshape(-1).astype(jnp.int32)
    out = _build(n)(W_word, W_pos, iw, ip)
    return out.reshape(B, L, DIM)


# speedup vs baseline: 1.9944x; 1.0051x over previous
import functools

import jax
import jax.numpy as jnp
from jax import lax
from jax.experimental import pallas as pl
from jax.experimental.pallas import tpu as pltpu
from jax.experimental.pallas import tpu_sc as plsc

DIM = 64
SCALE = 8.0
LANES = 16
CHUNK = 128


@functools.lru_cache(maxsize=None)
def _build(n_rows: int):
    info = plsc.get_sparse_core_info()
    nw = info.num_cores * info.num_subcores
    rows_per_w = n_rows // nw
    n_chunks = rows_per_w // CHUNK
    assert rows_per_w % CHUNK == 0 and n_chunks % 2 == 0

    mesh = plsc.VectorSubcoreMesh(core_axis_name="c", subcore_axis_name="s")

    @functools.partial(
        pl.kernel,
        mesh=mesh,
        compiler_params=pltpu.CompilerParams(use_tc_tiling_on_sc=False),
        out_type=jax.ShapeDtypeStruct((n_rows, DIM), jnp.float32),
        scratch_types=[
            pltpu.VMEM((rows_per_w,), jnp.int32),
            pltpu.VMEM((rows_per_w,), jnp.int32),
            pltpu.VMEM((CHUNK, DIM), jnp.float32),
            pltpu.VMEM((CHUNK, DIM), jnp.float32),
            pltpu.VMEM((CHUNK, DIM), jnp.float32),
            pltpu.VMEM((CHUNK, DIM), jnp.float32),
            pltpu.SemaphoreType.DMA,
            pltpu.SemaphoreType.DMA,
            pltpu.SemaphoreType.DMA,
            pltpu.SemaphoreType.DMA,
            pltpu.SemaphoreType.DMA,
            pltpu.SemaphoreType.DMA,
        ],
    )
    def emb(w_word, w_pos, iw_hbm, ip_hbm, out_hbm,
            iw_v, ip_v, bw0, bp0, bw1, bp1, sw0, sp0, sw1, sp1, so0, so1):
        wid = lax.axis_index("s") * info.num_cores + lax.axis_index("c")
        base = wid * rows_per_w
        pltpu.sync_copy(iw_hbm.at[pl.ds(base, rows_per_w)], iw_v)
        pltpu.sync_copy(ip_hbm.at[pl.ds(base, rows_per_w)], ip_v)

        bufs = ((bw0, bp0, sw0, sp0, so0), (bw1, bp1, sw1, sp1, so1))

        def issue(c, slot):
            bw, bp, sw, sp, _ = slot
            off = c * CHUNK
            pltpu.async_copy(w_word.at[iw_v.at[pl.ds(off, CHUNK)]], bw, sw)
            pltpu.async_copy(w_pos.at[ip_v.at[pl.ds(off, CHUNK)]], bp, sp)

        def wait_gathers(slot):
            bw, bp, sw, sp, _ = slot
            pltpu.make_async_copy(w_word.at[iw_v.at[pl.ds(0, CHUNK)]], bw, sw).wait()
            pltpu.make_async_copy(w_pos.at[ip_v.at[pl.ds(0, CHUNK)]], bp, sp).wait()

        def compute(slot):
            bw, bp = slot[0], slot[1]

            def row_body(r, carry):
                for k in range(4):
                    ri = r * 4 + k
                    for j in range(DIM // LANES):
                        s = pl.ds(j * LANES, LANES)
                        plsc.addupdate(bp.at[ri, s], bw[ri, s] * SCALE)
                return carry

            lax.fori_loop(0, CHUNK // 4, row_body, 0, unroll=False)

        def write(c, slot):
            bp, so = slot[1], slot[4]
            pltpu.async_copy(bp, out_hbm.at[pl.ds(base + c * CHUNK, CHUNK)], so)

        def wait_write(slot):
            bp, so = slot[1], slot[4]
            pltpu.make_async_copy(bp, out_hbm.at[pl.ds(base, CHUNK)], so).wait()

        issue(0, bufs[0])
        issue(1, bufs[1])
        wait_gathers(bufs[0])
        compute(bufs[0])
        write(0, bufs[0])
        wait_write(bufs[0])
        issue(2, bufs[0])
        wait_gathers(bufs[1])
        compute(bufs[1])
        write(1, bufs[1])

        def group(g, carry):
            for b in (0, 1):
                c = 2 * g + b
                sl = bufs[b]
                other = bufs[1 - b]

                @pl.when(c + 1 < n_chunks)
                def _():
                    wait_write(other)
                    issue(c + 1, other)

                wait_gathers(sl)
                compute(sl)
                write(c, sl)
            return carry

        lax.fori_loop(1, n_chunks // 2, group, 0, unroll=False)
        wait_write(bufs[0])
        wait_write(bufs[1])

    return emb


def kernel(src_word, src_pos, W_word, W_pos):
    B, L = src_word.shape
    n = B * L
    iw = src_word.reshape(-1).astype(jnp.int32)
    ip = src_pos.reshape(-1).astype(jnp.int32)
    out = _build(n)(W_word, W_pos, iw, ip)
    return out.reshape(B, L, DIM)


# speedup vs baseline: 1.9953x; 1.0004x over previous
import functools

import jax
import jax.numpy as jnp
from jax import lax
from jax.experimental import pallas as pl
from jax.experimental.pallas import tpu as pltpu
from jax.experimental.pallas import tpu_sc as plsc

DIM = 64
SCALE = 8.0
LANES = 16
CHUNK = 128


@functools.lru_cache(maxsize=None)
def _build(n_rows: int):
    info = plsc.get_sparse_core_info()
    nw = info.num_cores * info.num_subcores
    rows_per_w = n_rows // nw
    n_chunks = rows_per_w // CHUNK
    assert rows_per_w % CHUNK == 0 and n_chunks >= 4 and (n_chunks - 4) % 4 == 0

    mesh = plsc.VectorSubcoreMesh(core_axis_name="c", subcore_axis_name="s")

    @functools.partial(
        pl.kernel,
        mesh=mesh,
        compiler_params=pltpu.CompilerParams(use_tc_tiling_on_sc=False),
        out_type=jax.ShapeDtypeStruct((n_rows, DIM), jnp.float32),
        scratch_types=(
            [pltpu.VMEM((rows_per_w,), jnp.int32)] * 2
            + [pltpu.VMEM((CHUNK, DIM), jnp.float32)] * 8
            + [pltpu.SemaphoreType.DMA] * 12
        ),
    )
    def emb(w_word, w_pos, iw_hbm, ip_hbm, out_hbm, iw_v, ip_v, *rest):
        wid = lax.axis_index("s") * info.num_cores + lax.axis_index("c")
        base = wid * rows_per_w
        pltpu.sync_copy(iw_hbm.at[pl.ds(base, rows_per_w)], iw_v)
        pltpu.sync_copy(ip_hbm.at[pl.ds(base, rows_per_w)], ip_v)

        bufv, semv = rest[:8], rest[8:]
        bufs = tuple(
            (bufv[2 * i], bufv[2 * i + 1], semv[3 * i], semv[3 * i + 1], semv[3 * i + 2])
            for i in range(4)
        )

        def issue(c, slot):
            bw, bp, sw, sp, _ = slot
            off = c * CHUNK
            pltpu.async_copy(w_word.at[iw_v.at[pl.ds(off, CHUNK)]], bw, sw)
            pltpu.async_copy(w_pos.at[ip_v.at[pl.ds(off, CHUNK)]], bp, sp)

        def wait_gathers(slot):
            bw, bp, sw, sp, _ = slot
            pltpu.make_async_copy(w_word.at[iw_v.at[pl.ds(0, CHUNK)]], bw, sw).wait()
            pltpu.make_async_copy(w_pos.at[ip_v.at[pl.ds(0, CHUNK)]], bp, sp).wait()

        def compute(slot):
            bw, bp = slot[0], slot[1]

            def row_body(r, carry):
                for k in range(4):
                    ri = r * 4 + k
                    for j in range(DIM // LANES):
                        s = pl.ds(j * LANES, LANES)
                        plsc.addupdate(bp.at[ri, s], bw[ri, s] * SCALE)
                return carry

            lax.fori_loop(0, CHUNK // 4, row_body, 0, unroll=False)

        def write(c, slot):
            bp, so = slot[1], slot[4]
            pltpu.async_copy(bp, out_hbm.at[pl.ds(base + c * CHUNK, CHUNK)], so)

        def wait_write(slot):
            bp, so = slot[1], slot[4]
            pltpu.make_async_copy(bp, out_hbm.at[pl.ds(base, CHUNK)], so).wait()

        issue(0, bufs[0])
        issue(1, bufs[1])
        for c in (0, 1):
            issue(c + 2, bufs[c + 2])
            wait_gathers(bufs[c])
            compute(bufs[c])
            write(c, bufs[c])

        def group(g, carry):
            for k in range(4):
                c = 2 + 4 * g + k
                sl = bufs[(2 + k) % 4]
                nxt = bufs[k % 4]
                wait_write(nxt)
                issue(c + 2, nxt)
                wait_gathers(sl)
                compute(sl)
                write(c, sl)
            return carry

        lax.fori_loop(0, (n_chunks - 4) // 4, group, 0, unroll=False)
        for c in (n_chunks - 2, n_chunks - 1):
            sl = bufs[c % 4]
            wait_gathers(sl)
            compute(sl)
            write(c, sl)
        for b in range(4):
            wait_write(bufs[b])

    return emb


def kernel(src_word, src_pos, W_word, W_pos):
    B, L = src_word.shape
    n = B * L
    iw = src_word.reshape(-1).astype(jnp.int32)
    ip = src_pos.reshape(-1).astype(jnp.int32)
    out = _build(n)(W_word, W_pos, iw, ip)
    return out.reshape(B, L, DIM)


# speedup vs baseline: 1.9983x; 1.0015x over previous
import functools

import jax
import jax.numpy as jnp
from jax import lax
from jax.experimental import pallas as pl
from jax.experimental.pallas import tpu as pltpu
from jax.experimental.pallas import tpu_sc as plsc

DIM = 64
SCALE = 8.0
LANES = 16
CHUNK = 128


@functools.lru_cache(maxsize=None)
def _build(n_rows: int):
    info = plsc.get_sparse_core_info()
    nw = info.num_cores * info.num_subcores
    rows_per_w = n_rows // nw
    n_chunks = rows_per_w // CHUNK
    assert rows_per_w % CHUNK == 0 and n_chunks >= 4 and (n_chunks - 4) % 4 == 0

    mesh = plsc.VectorSubcoreMesh(core_axis_name="c", subcore_axis_name="s")

    @functools.partial(
        pl.kernel,
        mesh=mesh,
        compiler_params=pltpu.CompilerParams(use_tc_tiling_on_sc=False),
        out_type=jax.ShapeDtypeStruct((n_rows, DIM), jnp.float32),
        scratch_types=(
            [pltpu.VMEM((rows_per_w,), jnp.int32)] * 2
            + [pltpu.VMEM((CHUNK, DIM), jnp.float32)] * 4
            + [pltpu.SemaphoreType.DMA] * 12
        ),
    )
    def emb(w_word, w_pos, iw_hbm, ip_hbm, out_hbm, iw_v, ip_v, *rest):
        wid = lax.axis_index("s") * info.num_cores + lax.axis_index("c")
        base = wid * rows_per_w
        pltpu.sync_copy(iw_hbm.at[pl.ds(base, rows_per_w)], iw_v)
        pltpu.sync_copy(ip_hbm.at[pl.ds(base, rows_per_w)], ip_v)

        bufv, semv = rest[:4], rest[4:]
        bufs = tuple(
            (bufv[i], semv[3 * i], semv[3 * i + 1], semv[3 * i + 2]) for i in range(4)
        )

        def issue_word(c, slot):
            bw, sw, _, _ = slot
            pltpu.async_copy(w_word.at[iw_v.at[pl.ds(c * CHUNK, CHUNK)]], bw, sw)

        def wait_word(slot):
            bw, sw, _, _ = slot
            pltpu.make_async_copy(w_word.at[iw_v.at[pl.ds(0, CHUNK)]], bw, sw).wait()

        def scale(slot):
            bw = slot[0]

            def row_body(r, carry):
                for k in range(4):
                    ri = r * 4 + k
                    for j in range(DIM // LANES):
                        s = pl.ds(j * LANES, LANES)
                        bw[ri, s] = bw[ri, s] * SCALE
                return carry

            lax.fori_loop(0, CHUNK // 4, row_body, 0, unroll=False)

        def issue_posadd(c, slot):
            bw, _, sp, _ = slot
            pltpu.async_copy(
                w_pos.at[ip_v.at[pl.ds(c * CHUNK, CHUNK)]], bw, sp, add=True
            )

        def wait_posadd(slot):
            bw, _, sp, _ = slot
            pltpu.make_async_copy(w_pos.at[ip_v.at[pl.ds(0, CHUNK)]], bw, sp).wait()

        def write(c, slot):
            bw, _, _, so = slot
            pltpu.async_copy(bw, out_hbm.at[pl.ds(base + c * CHUNK, CHUNK)], so)

        def wait_write(slot):
            bw, _, _, so = slot
            pltpu.make_async_copy(bw, out_hbm.at[pl.ds(base, CHUNK)], so).wait()

        issue_word(0, bufs[0])
        issue_word(1, bufs[1])
        for c in (0, 1):
            if c > 0:
                wait_posadd(bufs[c - 1])
                write(c - 1, bufs[c - 1])
            issue_word(c + 2, bufs[c + 2])
            wait_word(bufs[c])
            scale(bufs[c])
            issue_posadd(c, bufs[c])

        def group(g, carry):
            for k in range(4):
                c = 2 + 4 * g + k
                prev = bufs[(1 + k) % 4]
                nxt = bufs[k % 4]
                cur = bufs[(2 + k) % 4]
                wait_posadd(prev)
                write(c - 1, prev)
                wait_write(nxt)
                issue_word(c + 2, nxt)
                wait_word(cur)
                scale(cur)
                issue_posadd(c, cur)
            return carry

        lax.fori_loop(0, (n_chunks - 4) // 4, group, 0, unroll=False)
        for c in (n_chunks - 2, n_chunks - 1):
            prev, cur = bufs[(c - 1) % 4], bufs[c % 4]
            wait_posadd(prev)
            write(c - 1, prev)
            wait_word(cur)
            scale(cur)
            issue_posadd(c, cur)
        last = bufs[(n_chunks - 1) % 4]
        wait_posadd(last)
        write(n_chunks - 1, last)
        for b in range(4):
            wait_write(bufs[b])

    return emb


def kernel(src_word, src_pos, W_word, W_pos):
    B, L = src_word.shape
    n = B * L
    iw = src_word.reshape(-1).astype(jnp.int32)
    ip = src_pos.reshape(-1).astype(jnp.int32)
    out = _build(n)(W_word, W_pos, iw, ip)
    return out.reshape(B, L, DIM)


# speedup vs baseline: 2.0020x; 1.0019x over previous
import functools

import jax
import jax.numpy as jnp
from jax import lax
from jax.experimental import pallas as pl
from jax.experimental.pallas import tpu as pltpu
from jax.experimental.pallas import tpu_sc as plsc

DIM = 64
SCALE = 8.0
LANES = 16
CHUNK = 256


@functools.lru_cache(maxsize=None)
def _build(n_rows: int):
    info = plsc.get_sparse_core_info()
    nw = info.num_cores * info.num_subcores
    rows_per_w = n_rows // nw
    n_chunks = rows_per_w // CHUNK
    assert rows_per_w % CHUNK == 0 and n_chunks >= 4 and (n_chunks - 4) % 4 == 0

    mesh = plsc.VectorSubcoreMesh(core_axis_name="c", subcore_axis_name="s")

    @functools.partial(
        pl.kernel,
        mesh=mesh,
        compiler_params=pltpu.CompilerParams(use_tc_tiling_on_sc=False),
        out_type=jax.ShapeDtypeStruct((n_rows, DIM), jnp.float32),
        scratch_types=(
            [pltpu.VMEM((rows_per_w,), jnp.int32)] * 2
            + [pltpu.VMEM((CHUNK, DIM), jnp.float32)] * 4
            + [pltpu.SemaphoreType.DMA] * 12
        ),
    )
    def emb(w_word, w_pos, iw_hbm, ip_hbm, out_hbm, iw_v, ip_v, *rest):
        wid = lax.axis_index("s") * info.num_cores + lax.axis_index("c")
        base = wid * rows_per_w
        pltpu.sync_copy(iw_hbm.at[pl.ds(base, rows_per_w)], iw_v)
        pltpu.sync_copy(ip_hbm.at[pl.ds(base, rows_per_w)], ip_v)

        bufv, semv = rest[:4], rest[4:]
        bufs = tuple(
            (bufv[i], semv[3 * i], semv[3 * i + 1], semv[3 * i + 2]) for i in range(4)
        )

        def issue_word(c, slot):
            bw, sw, _, _ = slot
            pltpu.async_copy(w_word.at[iw_v.at[pl.ds(c * CHUNK, CHUNK)]], bw, sw)

        def wait_word(slot):
            bw, sw, _, _ = slot
            pltpu.make_async_copy(w_word.at[iw_v.at[pl.ds(0, CHUNK)]], bw, sw).wait()

        def scale(slot):
            bw = slot[0]

            def row_body(r, carry):
                for k in range(4):
                    ri = r * 4 + k
                    for j in range(DIM // LANES):
                        s = pl.ds(j * LANES, LANES)
                        bw[ri, s] = bw[ri, s] * SCALE
                return carry

            lax.fori_loop(0, CHUNK // 4, row_body, 0, unroll=False)

        def issue_posadd(c, slot):
            bw, _, sp, _ = slot
            pltpu.async_copy(
                w_pos.at[ip_v.at[pl.ds(c * CHUNK, CHUNK)]], bw, sp, add=True
            )

        def wait_posadd(slot):
            bw, _, sp, _ = slot
            pltpu.make_async_copy(w_pos.at[ip_v.at[pl.ds(0, CHUNK)]], bw, sp).wait()

        def write(c, slot):
            bw, _, _, so = slot
            pltpu.async_copy(bw, out_hbm.at[pl.ds(base + c * CHUNK, CHUNK)], so)

        def wait_write(slot):
            bw, _, _, so = slot
            pltpu.make_async_copy(bw, out_hbm.at[pl.ds(base, CHUNK)], so).wait()

        issue_word(0, bufs[0])
        issue_word(1, bufs[1])
        for c in (0, 1):
            if c > 0:
                wait_posadd(bufs[c - 1])
                write(c - 1, bufs[c - 1])
            issue_word(c + 2, bufs[c + 2])
            wait_word(bufs[c])
            scale(bufs[c])
            issue_posadd(c, bufs[c])

        def group(g, carry):
            for k in range(4):
                c = 2 + 4 * g + k
                prev = bufs[(1 + k) % 4]
                nxt = bufs[k % 4]
                cur = bufs[(2 + k) % 4]
                wait_posadd(prev)
                write(c - 1, prev)
                wait_write(nxt)
                issue_word(c + 2, nxt)
                wait_word(cur)
                scale(cur)
                issue_posadd(c, cur)
            return carry

        lax.fori_loop(0, (n_chunks - 4) // 4, group, 0, unroll=False)
        for c in (n_chunks - 2, n_chunks - 1):
            prev, cur = bufs[(c - 1) % 4], bufs[c % 4]
            wait_posadd(prev)
            write(c - 1, prev)
            wait_word(cur)
            scale(cur)
            issue_posadd(c, cur)
        last = bufs[(n_chunks - 1) % 4]
        wait_posadd(last)
        write(n_chunks - 1, last)
        for b in range(4):
            wait_write(bufs[b])

    return emb


def kernel(src_word, src_pos, W_word, W_pos):
    B, L = src_word.shape
    n = B * L
    iw = src_word.reshape(-1).astype(jnp.int32)
    ip = src_pos.reshape(-1).astype(jnp.int32)
    out = _build(n)(W_word, W_pos, iw, ip)
    return out.reshape(B, L, DIM)


# speedup vs baseline: 2.6036x; 1.3005x over previous
import functools

import jax
import jax.numpy as jnp
from jax import lax
from jax.experimental import pallas as pl
from jax.experimental.pallas import tpu as pltpu
from jax.experimental.pallas import tpu_sc as plsc

DIM = 64
SCALE = 8.0
LANES = 16
CHUNK = 256
MAXPOS = 200


@functools.lru_cache(maxsize=None)
def _build(n_rows: int):
    info = plsc.get_sparse_core_info()
    nw = info.num_cores * info.num_subcores
    rows_per_w = n_rows // nw
    n_chunks = rows_per_w // CHUNK
    assert rows_per_w % CHUNK == 0 and n_chunks >= 4 and (n_chunks - 4) % 4 == 0

    mesh = plsc.VectorSubcoreMesh(core_axis_name="c", subcore_axis_name="s")

    @functools.partial(
        pl.kernel,
        mesh=mesh,
        compiler_params=pltpu.CompilerParams(use_tc_tiling_on_sc=False),
        out_type=jax.ShapeDtypeStruct((n_rows, DIM), jnp.float32),
        scratch_types=(
            [pltpu.VMEM((rows_per_w,), jnp.int32)] * 2
            + [pltpu.VMEM((CHUNK, DIM), jnp.float32)] * 4
            + [pltpu.SemaphoreType.DMA] * 12
        ),
    )
    def emb(w_word, w_pos, iw_hbm, ip_hbm, out_hbm, iw_v, ip_v, *rest):
        wid = lax.axis_index("s") * info.num_cores + lax.axis_index("c")
        base = wid * rows_per_w
        pltpu.sync_copy(iw_hbm.at[pl.ds(base, rows_per_w)], iw_v)
        pltpu.sync_copy(ip_hbm.at[pl.ds(base, rows_per_w)], ip_v)

        bufv, semv = rest[:4], rest[4:]
        bufs = tuple(
            (bufv[i], semv[3 * i], semv[3 * i + 1], semv[3 * i + 2]) for i in range(4)
        )

        def issue_word(c, slot):
            bw, sw, _, _ = slot
            pltpu.async_copy(w_word.at[iw_v.at[pl.ds(c * CHUNK, CHUNK)]], bw, sw)

        def wait_word(slot):
            bw, sw, _, _ = slot
            pltpu.make_async_copy(w_word.at[iw_v.at[pl.ds(0, CHUNK)]], bw, sw).wait()

        def scale(slot):
            bw = slot[0]

            def row_body(r, carry):
                for k in range(4):
                    ri = r * 4 + k
                    for j in range(DIM // LANES):
                        s = pl.ds(j * LANES, LANES)
                        bw[ri, s] = bw[ri, s] * SCALE
                return carry

            lax.fori_loop(0, CHUNK // 4, row_body, 0, unroll=False)

        def issue_posadd(c, slot):
            bw, _, sp, _ = slot
            pltpu.async_copy(
                w_pos.at[ip_v.at[pl.ds(c * CHUNK, CHUNK)]], bw, sp, add=True
            )

        def wait_posadd(slot):
            bw, _, sp, _ = slot
            pltpu.make_async_copy(w_pos.at[ip_v.at[pl.ds(0, CHUNK)]], bw, sp).wait()

        def write(c, slot):
            bw, _, _, so = slot
            pltpu.async_copy(bw, out_hbm.at[pl.ds(base + c * CHUNK, CHUNK)], so)

        def wait_write(slot):
            bw, _, _, so = slot
            pltpu.make_async_copy(bw, out_hbm.at[pl.ds(base, CHUNK)], so).wait()

        issue_word(0, bufs[0])
        issue_word(1, bufs[1])
        for c in (0, 1):
            if c > 0:
                wait_posadd(bufs[c - 1])
                write(c - 1, bufs[c - 1])
            issue_word(c + 2, bufs[c + 2])
            wait_word(bufs[c])
            scale(bufs[c])
            issue_posadd(c, bufs[c])

        def group(g, carry):
            for k in range(4):
                c = 2 + 4 * g + k
                prev = bufs[(1 + k) % 4]
                nxt = bufs[k % 4]
                cur = bufs[(2 + k) % 4]
                wait_posadd(prev)
                write(c - 1, prev)
                wait_write(nxt)
                issue_word(c + 2, nxt)
                wait_word(cur)
                scale(cur)
                issue_posadd(c, cur)
            return carry

        lax.fori_loop(0, (n_chunks - 4) // 4, group, 0, unroll=False)
        for c in (n_chunks - 2, n_chunks - 1):
            prev, cur = bufs[(c - 1) % 4], bufs[c % 4]
            wait_posadd(prev)
            write(c - 1, prev)
            wait_word(cur)
            scale(cur)
            issue_posadd(c, cur)
        last = bufs[(n_chunks - 1) % 4]
        wait_posadd(last)
        write(n_chunks - 1, last)
        for b in range(4):
            wait_write(bufs[b])

    return emb


def kernel(src_word, src_pos, W_word, W_pos):
    B, L = src_word.shape
    n = B * L
    nw = 32
    iw = src_word.reshape(-1).astype(jnp.int32)
    ip = src_pos.reshape(-1).astype(jnp.int32)
    w_pos_rep = jnp.tile(W_pos, (nw, 1))
    ip = ip + (jnp.arange(n, dtype=jnp.int32) // (n // nw)) * MAXPOS
    out = _build(n)(W_word, w_pos_rep, iw, ip)
    return out.reshape(B, L, DIM)


# speedup vs baseline: 2.6739x; 1.0270x over previous
import functools

import jax
import jax.numpy as jnp
from jax import lax
from jax.experimental import pallas as pl
from jax.experimental.pallas import tpu as pltpu
from jax.experimental.pallas import tpu_sc as plsc

DIM = 64
SCALE = 8.0
LANES = 16
CHUNK = 256
MAXPOS = 200


VOCAB_BLOCK = 2048


@functools.lru_cache(maxsize=None)
def _linearize_table(vocab: int, dim: int):
    nb = (vocab + VOCAB_BLOCK - 1) // VOCAB_BLOCK

    def body(wt_ref, out_ref):
        t = jnp.transpose(wt_ref[...], (1, 0))
        t3 = t.reshape(VOCAB_BLOCK // 2, 2, dim)
        out_ref[...] = jnp.concatenate([t3[:, 0, :], t3[:, 1, :]], axis=1)

    return pl.pallas_call(
        body,
        grid=(nb,),
        in_specs=[pl.BlockSpec((dim, VOCAB_BLOCK), lambda i: (0, i))],
        out_specs=pl.BlockSpec((VOCAB_BLOCK // 2, 2 * dim), lambda i: (i, 0)),
        out_shape=jax.ShapeDtypeStruct((vocab // 2, 2 * dim), jnp.float32),
    )


@functools.lru_cache(maxsize=None)
def _build(n_rows: int):
    info = plsc.get_sparse_core_info()
    nw = info.num_cores * info.num_subcores
    rows_per_w = n_rows // nw
    n_chunks = rows_per_w // CHUNK
    assert rows_per_w % CHUNK == 0 and n_chunks >= 4 and (n_chunks - 4) % 4 == 0

    mesh = plsc.VectorSubcoreMesh(core_axis_name="c", subcore_axis_name="s")

    @functools.partial(
        pl.kernel,
        mesh=mesh,
        compiler_params=pltpu.CompilerParams(use_tc_tiling_on_sc=False),
        out_type=jax.ShapeDtypeStruct((n_rows, DIM), jnp.float32),
        scratch_types=(
            [pltpu.VMEM((rows_per_w,), jnp.int32)] * 2
            + [pltpu.VMEM((CHUNK, DIM), jnp.float32)] * 4
            + [pltpu.SemaphoreType.DMA] * 12
        ),
    )
    def emb(w_word, w_pos, iw_hbm, ip_hbm, out_hbm, iw_v, ip_v, *rest):
        wid = lax.axis_index("s") * info.num_cores + lax.axis_index("c")
        base = wid * rows_per_w
        pltpu.sync_copy(iw_hbm.at[pl.ds(base, rows_per_w)], iw_v)
        pltpu.sync_copy(ip_hbm.at[pl.ds(base, rows_per_w)], ip_v)

        bufv, semv = rest[:4], rest[4:]
        bufs = tuple(
            (bufv[i], semv[3 * i], semv[3 * i + 1], semv[3 * i + 2]) for i in range(4)
        )

        def issue_word(c, slot):
            bw, sw, _, _ = slot
            pltpu.async_copy(w_word.at[iw_v.at[pl.ds(c * CHUNK, CHUNK)]], bw, sw)

        def wait_word(slot):
            bw, sw, _, _ = slot
            pltpu.make_async_copy(w_word.at[iw_v.at[pl.ds(0, CHUNK)]], bw, sw).wait()

        def scale(slot):
            bw = slot[0]

            def row_body(r, carry):
                for k in range(4):
                    ri = r * 4 + k
                    for j in range(DIM // LANES):
                        s = pl.ds(j * LANES, LANES)
                        bw[ri, s] = bw[ri, s] * SCALE
                return carry

            lax.fori_loop(0, CHUNK // 4, row_body, 0, unroll=False)

        def issue_posadd(c, slot):
            bw, _, sp, _ = slot
            pltpu.async_copy(
                w_pos.at[ip_v.at[pl.ds(c * CHUNK, CHUNK)]], bw, sp, add=True
            )

        def wait_posadd(slot):
            bw, _, sp, _ = slot
            pltpu.make_async_copy(w_pos.at[ip_v.at[pl.ds(0, CHUNK)]], bw, sp).wait()

        def write(c, slot):
            bw, _, _, so = slot
            pltpu.async_copy(bw, out_hbm.at[pl.ds(base + c * CHUNK, CHUNK)], so)

        def wait_write(slot):
            bw, _, _, so = slot
            pltpu.make_async_copy(bw, out_hbm.at[pl.ds(base, CHUNK)], so).wait()

        issue_word(0, bufs[0])
        issue_word(1, bufs[1])
        for c in (0, 1):
            if c > 0:
                wait_posadd(bufs[c - 1])
                write(c - 1, bufs[c - 1])
            issue_word(c + 2, bufs[c + 2])
            wait_word(bufs[c])
            scale(bufs[c])
            issue_posadd(c, bufs[c])

        def group(g, carry):
            for k in range(4):
                c = 2 + 4 * g + k
                prev = bufs[(1 + k) % 4]
                nxt = bufs[k % 4]
                cur = bufs[(2 + k) % 4]
                wait_posadd(prev)
                write(c - 1, prev)
                wait_write(nxt)
                issue_word(c + 2, nxt)
                wait_word(cur)
                scale(cur)
                issue_posadd(c, cur)
            return carry

        lax.fori_loop(0, (n_chunks - 4) // 4, group, 0, unroll=False)
        for c in (n_chunks - 2, n_chunks - 1):
            prev, cur = bufs[(c - 1) % 4], bufs[c % 4]
            wait_posadd(prev)
            write(c - 1, prev)
            wait_word(cur)
            scale(cur)
            issue_posadd(c, cur)
        last = bufs[(n_chunks - 1) % 4]
        wait_posadd(last)
        write(n_chunks - 1, last)
        for b in range(4):
            wait_write(bufs[b])

    return emb


def kernel(src_word, src_pos, W_word, W_pos):
    B, L = src_word.shape
    n = B * L
    nw = 32
    iw = src_word.reshape(-1).astype(jnp.int32)
    ip = src_pos.reshape(-1).astype(jnp.int32)
    w_pos_rep = jnp.tile(W_pos, (nw, 1))
    ip = ip + (jnp.arange(n, dtype=jnp.int32) // (n // nw)) * MAXPOS
    V = W_word.shape[0]
    w_lin = _linearize_table(V, DIM)(W_word.T).reshape(V, DIM)
    out = _build(n)(w_lin, w_pos_rep, iw, ip)
    return out.reshape(B, L, DIM)


# speedup vs baseline: 3.5849x; 1.3407x over previous
import functools

import jax
import jax.numpy as jnp
from jax import lax
from jax.experimental import pallas as pl
from jax.experimental.pallas import tpu as pltpu
from jax.experimental.pallas import tpu_sc as plsc

DIM = 64
SCALE = 8.0
LANES = 16
CHUNK = 256
MAXPOS = 200


VOCAB_BLOCK = 2048


@functools.lru_cache(maxsize=None)
def _linearize_table(vocab: int, dim: int):
    nb = (vocab + VOCAB_BLOCK - 1) // VOCAB_BLOCK

    def body(wt_ref, out_ref):
        t = jnp.transpose(wt_ref[...], (1, 0))
        t3 = t.reshape(VOCAB_BLOCK // 2, 2, dim)
        out_ref[...] = jnp.concatenate([t3[:, 0, :], t3[:, 1, :]], axis=1)

    return pl.pallas_call(
        body,
        grid=(nb,),
        in_specs=[pl.BlockSpec((dim, VOCAB_BLOCK), lambda i: (0, i))],
        out_specs=pl.BlockSpec((VOCAB_BLOCK // 2, 2 * dim), lambda i: (i, 0)),
        out_shape=jax.ShapeDtypeStruct((vocab // 2, 2 * dim), jnp.float32),
    )


@functools.lru_cache(maxsize=None)
def _build(n_rows: int):
    info = plsc.get_sparse_core_info()
    nw = info.num_cores * info.num_subcores
    rows_per_w = n_rows // nw
    n_chunks = rows_per_w // CHUNK
    assert rows_per_w % CHUNK == 0 and n_chunks >= 4 and (n_chunks - 4) % 4 == 0

    mesh = plsc.VectorSubcoreMesh(core_axis_name="c", subcore_axis_name="s")

    @functools.partial(
        pl.kernel,
        mesh=mesh,
        compiler_params=pltpu.CompilerParams(use_tc_tiling_on_sc=False),
        out_type=jax.ShapeDtypeStruct((n_rows, 2 * DIM), jnp.float32),
        scratch_types=(
            [pltpu.VMEM((rows_per_w,), jnp.int32)] * 2
            + [pltpu.VMEM((CHUNK, DIM), jnp.float32)] * 4
            + [pltpu.SemaphoreType.DMA] * 12
        ),
    )
    def emb(w_word, w_pos, iw_hbm, ip_hbm, out_hbm, iw_v, ip_v, *rest):
        wid = lax.axis_index("s") * info.num_cores + lax.axis_index("c")
        base = wid * rows_per_w
        pltpu.sync_copy(iw_hbm.at[pl.ds(base, rows_per_w)], iw_v)
        pltpu.sync_copy(ip_hbm.at[pl.ds(base, rows_per_w)], ip_v)

        bufv, semv = rest[:4], rest[4:]
        bufs = tuple(
            (bufv[i], semv[3 * i], semv[3 * i + 1], semv[3 * i + 2]) for i in range(4)
        )

        def issue_word(c, slot):
            bw, sw, _, _ = slot
            pltpu.async_copy(w_word.at[iw_v.at[pl.ds(c * CHUNK, CHUNK)]], bw, sw)

        def wait_word(slot):
            bw, sw, _, _ = slot
            pltpu.make_async_copy(w_word.at[iw_v.at[pl.ds(0, CHUNK)]], bw, sw).wait()

        def scale(slot):
            bw = slot[0]

            def row_body(r, carry):
                for k in range(4):
                    ri = r * 4 + k
                    for j in range(DIM // LANES):
                        s = pl.ds(j * LANES, LANES)
                        bw[ri, s] = bw[ri, s] * SCALE
                return carry

            lax.fori_loop(0, CHUNK // 4, row_body, 0, unroll=False)

        def issue_posadd(c, slot):
            bw, _, sp, _ = slot
            pltpu.async_copy(
                w_pos.at[ip_v.at[pl.ds(c * CHUNK, CHUNK)]], bw, sp, add=True
            )

        def wait_posadd(slot):
            bw, _, sp, _ = slot
            pltpu.make_async_copy(w_pos.at[ip_v.at[pl.ds(0, CHUNK)]], bw, sp).wait()

        def write(c, slot):
            bw, _, _, so = slot
            pltpu.async_copy(bw, out_hbm.at[pl.ds(base + c * CHUNK, CHUNK), pl.ds(0, DIM)], so)

        def wait_write(slot):
            bw, _, _, so = slot
            pltpu.make_async_copy(bw, out_hbm.at[pl.ds(base, CHUNK), pl.ds(0, DIM)], so).wait()

        issue_word(0, bufs[0])
        issue_word(1, bufs[1])
        for c in (0, 1):
            if c > 0:
                wait_posadd(bufs[c - 1])
                write(c - 1, bufs[c - 1])
            issue_word(c + 2, bufs[c + 2])
            wait_word(bufs[c])
            scale(bufs[c])
            issue_posadd(c, bufs[c])

        def group(g, carry):
            for k in range(4):
                c = 2 + 4 * g + k
                prev = bufs[(1 + k) % 4]
                nxt = bufs[k % 4]
                cur = bufs[(2 + k) % 4]
                wait_posadd(prev)
                write(c - 1, prev)
                wait_write(nxt)
                issue_word(c + 2, nxt)
                wait_word(cur)
                scale(cur)
                issue_posadd(c, cur)
            return carry

        lax.fori_loop(0, (n_chunks - 4) // 4, group, 0, unroll=False)
        for c in (n_chunks - 2, n_chunks - 1):
            prev, cur = bufs[(c - 1) % 4], bufs[c % 4]
            wait_posadd(prev)
            write(c - 1, prev)
            wait_word(cur)
            scale(cur)
            issue_posadd(c, cur)
        last = bufs[(n_chunks - 1) % 4]
        wait_posadd(last)
        write(n_chunks - 1, last)
        for b in range(4):
            wait_write(bufs[b])

    return emb


def kernel(src_word, src_pos, W_word, W_pos):
    B, L = src_word.shape
    n = B * L
    nw = 32
    iw = src_word.reshape(-1).astype(jnp.int32)
    ip = src_pos.reshape(-1).astype(jnp.int32)
    w_pos_rep = jnp.tile(W_pos, (nw, 1))
    ip = ip + (jnp.arange(n, dtype=jnp.int32) // (n // nw)) * MAXPOS
    V = W_word.shape[0]
    w_lin = _linearize_table(V, DIM)(W_word.T).reshape(V, DIM)
    out = _build(n)(w_lin, w_pos_rep, iw, ip)
    return out[:, :DIM].reshape(B, L, DIM)
